# Initial kernel scaffold; baseline (speedup 1.0000x reference)
import jax
import jax.numpy as jnp
from jax.experimental import pallas as pl


def kernel(xyz, features, sample_idx, group_idx, W1, b1, W2, b2):
    raise NotImplementedError("write your pallas kernel here")


# speedup vs baseline: 38.1563x; 38.1563x over previous
import functools

import jax
import jax.numpy as jnp
from jax import lax
from jax.experimental import pallas as pl
from jax.experimental.pallas import tpu as pltpu
from jax.experimental.pallas import tpu_sc as plsc

_NC = 2
_NSUB = 16
_NW = _NC * _NSUB

_B, _N, _C = 4, 16384, 16
_NP, _NS = 4096, 32
_ROWS = _B * _NP * _NS
_PER_W = _ROWS // _NW
_CH = 128
_NCH = _PER_W // _CH
_K = 8
_CTR = _B * _NP
_CTR_W = _CTR // _NW
_D = 32


def _sc_gather_body(table, gidx, cidx, g_out, c_out, idx_v, cidx_v, rows_v,
                    crows_v, sem):
    wid = lax.axis_index("s") * _NC + lax.axis_index("c")
    pltpu.sync_copy(gidx.at[wid], idx_v)
    pltpu.sync_copy(cidx.at[wid], cidx_v)

    nc_ch = _CTR_W // _CH
    hs = [pltpu.async_copy(table.at[cidx_v.at[j]],
                           crows_v.at[pl.ds(j * _CH, _CH)], sem)
          for j in range(nc_ch)]
    for h in hs:
        h.wait()
    pltpu.sync_copy(crows_v, c_out.at[pl.ds(wid * _CTR_W, _CTR_W)])

    def group(gi, carry):
        hs = [pltpu.async_copy(table.at[idx_v.at[gi * _K + k]],
                               rows_v.at[pl.ds(k * _CH, _CH)], sem)
              for k in range(_K)]
        for k in range(_K):
            hs[k].wait()
            pltpu.sync_copy(
                rows_v.at[pl.ds(k * _CH, _CH)],
                g_out.at[pl.ds((wid * _NCH + gi * _K + k) * _CH, _CH)])
        return carry

    lax.fori_loop(0, _NCH // _K, group, 0)


@functools.cache
def _sc_gather_kernel():
    return pl.kernel(
        _sc_gather_body,
        out_type=[
            jax.ShapeDtypeStruct((_ROWS, _D), jnp.float32),
            jax.ShapeDtypeStruct((_CTR, _D), jnp.float32),
        ],
        mesh=plsc.VectorSubcoreMesh(core_axis_name="c", subcore_axis_name="s"),
        scratch_types=[
            pltpu.VMEM((_NCH, _CH), jnp.int32),
            pltpu.VMEM((_CTR_W // _CH, _CH), jnp.int32),
            pltpu.VMEM((_K * _CH, _D), jnp.float32),
            pltpu.VMEM((_CTR_W, _D), jnp.float32),
            pltpu.SemaphoreType.DMA,
        ],
        compiler_params=pltpu.CompilerParams(use_tc_tiling_on_sc=False),
    )


_PBLK = 128
_RBLK = _PBLK * _NS


def _tc_mlp_body(g_ref, c_ref, w1p_ref, w1x_ref, b1_ref, w2_ref, b2_ref,
                 o_ref):
    g = g_ref[...]
    a = jnp.dot(g, w1p_ref[...],
                preferred_element_type=jnp.float32) + b1_ref[...]
    cm = jnp.dot(c_ref[...], w1x_ref[...],
                 preferred_element_type=jnp.float32)
    h1 = jnp.maximum(a.reshape(_PBLK, _NS, 32) - cm[:, None, :], 0.0)
    h2 = jnp.maximum(
        jnp.dot(h1.reshape(_RBLK, 32), w2_ref[...],
                preferred_element_type=jnp.float32) + b2_ref[...], 0.0)
    o_ref[...] = jnp.max(h2.reshape(_PBLK, _NS, 64), axis=1)


def _tc_mlp(g, ctr, w1p, w1x, b1r, w2, b2r):
    nblk = _CTR // _PBLK
    return pl.pallas_call(
        _tc_mlp_body,
        grid=(nblk,),
        in_specs=[
            pl.BlockSpec((_RBLK, _D), lambda i: (i, 0)),
            pl.BlockSpec((_PBLK, _D), lambda i: (i, 0)),
            pl.BlockSpec((_D, 32), lambda i: (0, 0)),
            pl.BlockSpec((_D, 32), lambda i: (0, 0)),
            pl.BlockSpec((1, 32), lambda i: (0, 0)),
            pl.BlockSpec((32, 64), lambda i: (0, 0)),
            pl.BlockSpec((1, 64), lambda i: (0, 0)),
        ],
        out_specs=pl.BlockSpec((_PBLK, 64), lambda i: (i, 0)),
        out_shape=jax.ShapeDtypeStruct((_CTR, 64), jnp.float32),
    )(g, ctr, w1p, w1x, b1r, w2, b2r)


def kernel(xyz, features, sample_idx, group_idx, W1, b1, W2, b2):
    B, N, _ = xyz.shape
    NP = sample_idx.shape[1]
    feats = jnp.transpose(features, (0, 2, 1))
    table = jnp.concatenate(
        [xyz, feats, jnp.zeros((B, N, _D - 3 - _C), jnp.float32)],
        axis=-1).reshape(B * N, _D)
    offs = (jnp.arange(B, dtype=jnp.int32) * N)
    gidx = (group_idx + offs[:, None, None]).reshape(_NW, _NCH, _CH)
    cidx = (sample_idx + offs[:, None]).reshape(_NW, _CTR_W // _CH, _CH)

    W1p = jnp.zeros((_D, 32), jnp.float32).at[:3 + _C].set(W1)
    W1x = jnp.zeros((_D, 32), jnp.float32).at[:3].set(W1[:3])

    g, ctr = _sc_gather_kernel()(table, gidx, cidx)
    out = _tc_mlp(g, ctr, W1p, W1x, b1.reshape(1, 32), W2, b2.reshape(1, 64))
    return out.reshape(B, NP, 64)


# speedup vs baseline: 75.0372x; 1.9666x over previous
import functools

import jax
import jax.numpy as jnp
from jax import lax
from jax.experimental import pallas as pl
from jax.experimental.pallas import tpu as pltpu
from jax.experimental.pallas import tpu_sc as plsc

_NC = 2
_NSUB = 16
_NW = _NC * _NSUB

_B, _N, _C = 4, 16384, 16
_NP, _NS = 4096, 32
_ROWS = _B * _NP * _NS
_PER_W = _ROWS // _NW
_CH = 128
_NCH = _PER_W // _CH
_K = 8
_CTR = _B * _NP
_REP = 4
_CCH = _CTR * _REP // _CH // _NW
_D = 32


def _sc_gather_body(table, gidx, cidx, g_out, c_out, idx_v, cidx_v, rows_v,
                    crows_v, sem):
    wid = lax.axis_index("s") * _NC + lax.axis_index("c")
    pltpu.sync_copy(gidx.at[wid], idx_v)
    pltpu.sync_copy(cidx.at[wid], cidx_v)

    def cgroup(gi, carry):
        hs = [pltpu.async_copy(table.at[cidx_v.at[gi * _K + k]],
                               crows_v.at[pl.ds(k * _CH, _CH)], sem)
              for k in range(_K)]
        for k in range(_K):
            hs[k].wait()
            pltpu.sync_copy(crows_v.at[pl.ds(k * _CH, _CH)],
                            c_out.at[wid * _CCH + gi * _K + k])
        return carry

    lax.fori_loop(0, _CCH // _K, cgroup, 0)

    def group(gi, carry):
        hs = [pltpu.async_copy(table.at[idx_v.at[gi * _K + k]],
                               rows_v.at[pl.ds(k * _CH, _CH)], sem)
              for k in range(_K)]
        for k in range(_K):
            hs[k].wait()
            pltpu.sync_copy(rows_v.at[pl.ds(k * _CH, _CH)],
                            g_out.at[wid * _NCH + gi * _K + k])
        return carry

    lax.fori_loop(0, _NCH // _K, group, 0)


@functools.cache
def _sc_gather_kernel():
    return pl.kernel(
        _sc_gather_body,
        out_type=[
            jax.ShapeDtypeStruct((_NW * _NCH, _CH, _D), jnp.float32),
            jax.ShapeDtypeStruct((_NW * _CCH, _CH, _D), jnp.float32),
        ],
        mesh=plsc.VectorSubcoreMesh(core_axis_name="c", subcore_axis_name="s"),
        scratch_types=[
            pltpu.VMEM((_NCH, _CH), jnp.int32),
            pltpu.VMEM((_CCH, _CH), jnp.int32),
            pltpu.VMEM((_K * _CH, _D), jnp.float32),
            pltpu.VMEM((_K * _CH, _D), jnp.float32),
            pltpu.SemaphoreType.DMA,
        ],
        compiler_params=pltpu.CompilerParams(use_tc_tiling_on_sc=False),
    )


_PBLK = 128
_GBLK = _PBLK * _NS // _REP


def _tc_mlp_body(g_ref, c_ref, w1_ref, w1x_ref, w2_ref, b1_ref, b2_ref,
                 o_ref):
    g = g_ref[...]
    a = jnp.dot(g, w1_ref[...],
                preferred_element_type=jnp.float32) + b1_ref[...]
    cm = jnp.dot(c_ref[...], w1x_ref[...],
                 preferred_element_type=jnp.float32)
    a3 = a.reshape(_PBLK, _NS // _REP, 128)
    h1 = jnp.maximum(a3 - cm[:, None, :], 0.0)
    h2 = jnp.maximum(
        jnp.dot(h1.reshape(_GBLK, 128), w2_ref[...],
                preferred_element_type=jnp.float32) + b2_ref[...], 0.0)
    m = jnp.max(h2.reshape(_PBLK, _NS // _REP, 256), axis=1)
    o_ref[...] = jnp.maximum(jnp.maximum(m[:, 0:64], m[:, 64:128]),
                             jnp.maximum(m[:, 128:192], m[:, 192:256]))


def _tc_mlp(g, ctr, w1bd, w1xbd, w2bd, b1t, b2t):
    nblk = _CTR // _PBLK
    return pl.pallas_call(
        _tc_mlp_body,
        grid=(nblk,),
        in_specs=[
            pl.BlockSpec((_GBLK, 128), lambda i: (i, 0)),
            pl.BlockSpec((_PBLK, 128), lambda i: (i, 0)),
            pl.BlockSpec((128, 128), lambda i: (0, 0)),
            pl.BlockSpec((128, 128), lambda i: (0, 0)),
            pl.BlockSpec((128, 256), lambda i: (0, 0)),
            pl.BlockSpec((1, 128), lambda i: (0, 0)),
            pl.BlockSpec((1, 256), lambda i: (0, 0)),
        ],
        out_specs=pl.BlockSpec((_PBLK, 64), lambda i: (i, 0)),
        out_shape=jax.ShapeDtypeStruct((_CTR, 64), jnp.float32),
    )(g, ctr, w1bd, w1xbd, w2bd, b1t, b2t)


def kernel(xyz, features, sample_idx, group_idx, W1, b1, W2, b2):
    B, N, _ = xyz.shape
    NP = sample_idx.shape[1]
    feats = jnp.transpose(features, (0, 2, 1))
    table = jnp.concatenate(
        [xyz, feats, jnp.zeros((B, N, _D - 3 - _C), jnp.float32)],
        axis=-1).reshape(B * N, _D)
    offs = (jnp.arange(B, dtype=jnp.int32) * N)
    gidx = (group_idx + offs[:, None, None]).reshape(_NW, _NCH, _CH)
    cidx = jnp.repeat((sample_idx + offs[:, None]).reshape(-1),
                      _REP).reshape(_NW, _CCH, _CH)

    W1p = jnp.zeros((_D, 32), jnp.float32).at[:3 + _C].set(W1)
    W1x = jnp.zeros((_D, 32), jnp.float32).at[:3].set(W1[:3])
    eye4 = jnp.eye(_REP, dtype=jnp.float32)
    W1bd = jnp.kron(eye4, W1p)
    W1xbd = jnp.kron(eye4, W1x)
    W2bd = jnp.kron(eye4, W2)
    b1t = jnp.tile(b1, _REP).reshape(1, 128)
    b2t = jnp.tile(b2, _REP).reshape(1, 256)

    g3, c3 = _sc_gather_kernel()(table, gidx, cidx)
    gp = g3.reshape(_ROWS // _REP, 128)
    cp = c3.reshape(_CTR, 128)
    out = _tc_mlp(gp, cp, W1bd, W1xbd, W2bd, b1t, b2t)
    return out.reshape(B, NP, 64)
